# baseline (speedup 1.0000x reference)
import functools

import numpy as np
import jax
import jax.numpy as jnp
from jax import lax
from jax.experimental import pallas as pl
from jax.experimental.pallas import tpu as pltpu
from jax.experimental.pallas import tpu_sc as plsc

_N_USERS = 100000
_N_ITEMS = 100000
_D = 128
_N_EDGES = 2000000
_ALPHA = 0.1
_K = 100000

_NW = 32
_CHUNK = 128
_NCH = 26
_NCH_PAD = 28
_PER_W = _NCH_PAD * _CHUNK
_VALID_W = _K // _NW
_NSPLIT = 2


def _build_sample_constants():
    with jax.ensure_compile_time_eval():
        skey = jax.random.key(42)
        perm = jax.random.permutation(jax.random.fold_in(skey, 0), _N_EDGES)[:_K]
        negj = jax.random.randint(jax.random.fold_in(skey, 1), (_K,), 1,
                                  _N_ITEMS + 1)
        perm = np.asarray(perm, dtype=np.int32)
        negj = np.asarray(negj, dtype=np.int32)
    order = np.argsort(perm)
    perm = perm[order]
    negj = negj[order]
    pm = np.zeros((_NW, _PER_W), np.int32)
    nj = np.ones((_NW, _PER_W), np.int32)
    pm[:, :_VALID_W] = perm.reshape(_NW, _VALID_W)
    nj[:, :_VALID_W] = negj.reshape(_NW, _VALID_W)
    return (pm.reshape(_NW, _NCH_PAD, _CHUNK),
            nj.reshape(_NW, _NCH_PAD, _CHUNK))


_CONSTS_CACHE = None


def _sample_constants():
    global _CONSTS_CACHE
    if _CONSTS_CACHE is None:
        try:
            _CONSTS_CACHE = _build_sample_constants()
        except Exception:
            return (np.zeros((_NW, _NCH_PAD, _CHUNK), np.int32),
                    np.ones((_NW, _NCH_PAD, _CHUNK), np.int32))
    return _CONSTS_CACHE


def _log_newton(x):
    bits = lax.bitcast_convert_type(x, jnp.int32)
    ln2_over_2_23 = float(np.log(2.0) / (1 << 23))
    offset = float(126.94269504 * np.log(2.0))
    y = bits.astype(jnp.float32) * ln2_over_2_23 - offset
    for _ in range(3):
        y = y + x * jnp.exp(-y) - 1.0
    return y


def _lane_total(v, mb):
    t = v
    for s in (8, 4, 2, 1):
        mb[pl.ds(0, 16)] = t
        t = t + mb[pl.ds(s, 16)]
    return t


def _sc_body(user_hbm, item_hbm, eu_hbm, ei_hbm, pm_hbm, nj_hbm, out_hbm,
             pm_v, nj_v, uidx_v, iidx_v,
             ur0, ur1, ir0, ir1, jr0, jr1,
             acc_v, mb_v, nb_v, pk_v, nk_v,
             semu, semi, *rsems):
    wid = lax.axis_index("s") * 2 + lax.axis_index("c")
    pltpu.sync_copy(pm_hbm.at[wid], pm_v)
    pltpu.sync_copy(nj_hbm.at[wid], nj_v)

    ucps = [pltpu.async_copy(eu_hbm.at[pm_v.at[k]], uidx_v.at[k], semu)
            for k in range(_NCH_PAD)]
    icps = [pltpu.async_copy(ei_hbm.at[pm_v.at[k]], iidx_v.at[k], semi)
            for k in range(_NCH_PAD)]
    for cp in ucps:
        cp.wait()
    for cp in icps:
        cp.wait()

    ur = (ur0, ur1)
    ir = (ir0, ir1)
    jr = (jr0, jr1)
    lane = lax.iota(jnp.int32, 16)
    half = _CHUNK // _NSPLIT

    def issue_rows(c, s):
        cps = []
        for q in range(_NSPLIT):
            sl = pl.ds(q * half, half)
            sb = 3 * _NSPLIT * s + 3 * q
            cps.append(pltpu.async_copy(
                user_hbm.at[uidx_v.at[c].at[sl]], ur[s].at[sl], rsems[sb]))
            cps.append(pltpu.async_copy(
                item_hbm.at[iidx_v.at[c].at[sl]], ir[s].at[sl], rsems[sb + 1]))
            cps.append(pltpu.async_copy(
                item_hbm.at[nj_v.at[c].at[sl]], jr[s].at[sl], rsems[sb + 2]))
        return cps

    def compute(c, s, acc):
        urows_v, irows_v, jrows_v = ur[s], ir[s], jr[s]

        def group_body(g, gacc):
            def edge_body(e, dummy):
                row = g * 16 + e
                u0 = urows_v[row, pl.ds(0, 16)]
                ap = u0 * irows_v[row, pl.ds(0, 16)]
                an = u0 * jrows_v[row, pl.ds(0, 16)]
                for d in range(1, 8):
                    ud = urows_v[row, pl.ds(16 * d, 16)]
                    ap = ap + ud * irows_v[row, pl.ds(16 * d, 16)]
                    an = an + ud * jrows_v[row, pl.ds(16 * d, 16)]
                pk_v[pl.ds(e, 16)] = _lane_total(ap, mb_v)
                nk_v[pl.ds(e, 16)] = _lane_total(an, nb_v)
                return dummy

            lax.fori_loop(0, 16, edge_body, jnp.int32(0), unroll=4)
            pvec = pk_v[pl.ds(0, 16)]
            nvec = nk_v[pl.ds(0, 16)]
            sp = 1.0 / (1.0 + jnp.exp(-pvec))
            lp = _log_newton(sp + 1e-10)
            sn = 1.0 / (1.0 + jnp.exp(nvec))
            ln_ = _log_newton(sn + 1e-10)
            gidx = c * _CHUNK + g * 16 + lane
            contrib = jnp.where(gidx < _VALID_W, lp + _ALPHA * ln_,
                                jnp.zeros((16,), jnp.float32))
            return gacc - contrib

        return lax.fori_loop(0, _CHUNK // 16, group_body, acc)

    for cp in issue_rows(0, 0):
        cp.wait()

    def pair_body(t, acc):
        a = 2 * t
        b = a + 1
        cps_b = issue_rows(b, 1)
        acc = compute(a, 0, acc)
        for cp in cps_b:
            cp.wait()
        cps_a2 = issue_rows(a + 2, 0)
        acc = compute(b, 1, acc)
        for cp in cps_a2:
            cp.wait()
        return acc

    acc = lax.fori_loop(0, _NCH // 2, pair_body,
                        jnp.zeros((16,), jnp.float32))

    acc_v[...] = acc
    pltpu.sync_copy(acc_v, out_hbm.at[wid])


@jax.jit
def _sc_loss(user_embs, item_embs, edge_u, edge_i, pm, nj):
    mesh = plsc.VectorSubcoreMesh(core_axis_name="c", subcore_axis_name="s")
    f = pl.kernel(
        _sc_body,
        out_type=jax.ShapeDtypeStruct((_NW, 16), jnp.float32),
        mesh=mesh,
        scratch_types=[
            pltpu.VMEM((_NCH_PAD, _CHUNK), jnp.int32),
            pltpu.VMEM((_NCH_PAD, _CHUNK), jnp.int32),
            pltpu.VMEM((_NCH_PAD, _CHUNK), jnp.int32),
            pltpu.VMEM((_NCH_PAD, _CHUNK), jnp.int32),
            pltpu.VMEM((_CHUNK, _D), jnp.float32),
            pltpu.VMEM((_CHUNK, _D), jnp.float32),
            pltpu.VMEM((_CHUNK, _D), jnp.float32),
            pltpu.VMEM((_CHUNK, _D), jnp.float32),
            pltpu.VMEM((_CHUNK, _D), jnp.float32),
            pltpu.VMEM((_CHUNK, _D), jnp.float32),
            pltpu.VMEM((16,), jnp.float32),
            pltpu.VMEM((32,), jnp.float32),
            pltpu.VMEM((32,), jnp.float32),
            pltpu.VMEM((32,), jnp.float32),
            pltpu.VMEM((32,), jnp.float32),
        ] + [pltpu.SemaphoreType.DMA] * (2 + 6 * _NSPLIT),
    )
    partials = f(user_embs, item_embs, edge_u, edge_i, pm, nj)
    return jnp.sum(partials)


def kernel(user_embs, item_embs, edge_u, edge_i):
    pm_np, nj_np = _sample_constants()
    pm = jnp.asarray(pm_np)
    nj = jnp.asarray(nj_np)
    return _sc_loss(user_embs, item_embs,
                    edge_u.astype(jnp.int32), edge_i.astype(jnp.int32),
                    pm, nj)


# speedup vs baseline: 2.1643x; 2.1643x over previous
import functools

import numpy as np
import jax
import jax.numpy as jnp
from jax import lax
from jax.experimental import pallas as pl
from jax.experimental.pallas import tpu as pltpu
from jax.experimental.pallas import tpu_sc as plsc

_N_USERS = 100000
_N_ITEMS = 100000
_D = 128
_N_EDGES = 2000000
_ALPHA = 0.1
_K = 100000

_NW = 32
_CHUNK = 128
_NCH = 25
_PER_W = _NCH * _CHUNK
_VALID_W = _K // _NW


def _build_sample_constants():
    with jax.ensure_compile_time_eval():
        skey = jax.random.key(42)
        perm = jax.random.permutation(jax.random.fold_in(skey, 0), _N_EDGES)[:_K]
        negj = jax.random.randint(jax.random.fold_in(skey, 1), (_K,), 1,
                                  _N_ITEMS + 1)
        perm = np.asarray(perm, dtype=np.int32)
        negj = np.asarray(negj, dtype=np.int32)
    order = np.argsort(perm)
    perm = perm[order]
    negj = negj[order]
    pm = np.zeros((_NW, _PER_W), np.int32)
    nj = np.ones((_NW, _PER_W), np.int32)
    pm[:, :_VALID_W] = perm.reshape(_NW, _VALID_W)
    nj[:, :_VALID_W] = negj.reshape(_NW, _VALID_W)
    return pm.reshape(_NW, _NCH, _CHUNK), nj.reshape(_NW, _NCH, _CHUNK)


_CONSTS_CACHE = None


def _sample_constants():
    global _CONSTS_CACHE
    if _CONSTS_CACHE is None:
        try:
            _CONSTS_CACHE = _build_sample_constants()
        except Exception:
            return (np.zeros((_NW, _NCH, _CHUNK), np.int32),
                    np.ones((_NW, _NCH, _CHUNK), np.int32))
    return _CONSTS_CACHE


def _log_newton(x):
    bits = lax.bitcast_convert_type(x, jnp.int32)
    ln2_over_2_23 = float(np.log(2.0) / (1 << 23))
    offset = float(126.94269504 * np.log(2.0))
    y = bits.astype(jnp.float32) * ln2_over_2_23 - offset
    for _ in range(3):
        y = y + x * jnp.exp(-y) - 1.0
    return y


def _lane_total(v, mb):
    t = v
    for s in (8, 4, 2, 1):
        mb[pl.ds(0, 16)] = t
        t = t + mb[pl.ds(s, 16)]
    return t


def _sc_body(user_hbm, item_hbm, eu_hbm, ei_hbm, pm_hbm, nj_hbm, out_hbm,
             pm_v, nj_v, uidx_v, iidx_v, urows_v, irows_v, jrows_v,
             acc_v, mb_v, nb_v, pk_v, nk_v,
             sem1, sem2, sem3, sem4, sem5, sem6, sem7, sem8):
    wid = lax.axis_index("s") * 2 + lax.axis_index("c")
    pltpu.sync_copy(pm_hbm.at[wid], pm_v)
    pltpu.sync_copy(nj_hbm.at[wid], nj_v)

    lane = lax.iota(jnp.int32, 16)

    def chunk_body(c, acc):
        cp_u = pltpu.async_copy(eu_hbm.at[pm_v.at[c]], uidx_v, sem1)
        cp_i = pltpu.async_copy(ei_hbm.at[pm_v.at[c]], iidx_v, sem2)
        cp_u.wait()
        cp_i.wait()
        cps = [
            pltpu.async_copy(user_hbm.at[uidx_v.at[pl.ds(0, 64)]],
                             urows_v.at[pl.ds(0, 64)], sem3),
            pltpu.async_copy(user_hbm.at[uidx_v.at[pl.ds(64, 64)]],
                             urows_v.at[pl.ds(64, 64)], sem6),
            pltpu.async_copy(item_hbm.at[iidx_v.at[pl.ds(0, 64)]],
                             irows_v.at[pl.ds(0, 64)], sem4),
            pltpu.async_copy(item_hbm.at[iidx_v.at[pl.ds(64, 64)]],
                             irows_v.at[pl.ds(64, 64)], sem7),
            pltpu.async_copy(item_hbm.at[nj_v.at[c].at[pl.ds(0, 64)]],
                             jrows_v.at[pl.ds(0, 64)], sem5),
            pltpu.async_copy(item_hbm.at[nj_v.at[c].at[pl.ds(64, 64)]],
                             jrows_v.at[pl.ds(64, 64)], sem8),
        ]
        for cp in cps:
            cp.wait()

        def group_body(g, gacc):
            for e in range(16):
                row = g * 16 + e
                u0 = urows_v[row, pl.ds(0, 16)]
                ap = u0 * irows_v[row, pl.ds(0, 16)]
                an = u0 * jrows_v[row, pl.ds(0, 16)]
                for d in range(1, 8):
                    ud = urows_v[row, pl.ds(16 * d, 16)]
                    ap = ap + ud * irows_v[row, pl.ds(16 * d, 16)]
                    an = an + ud * jrows_v[row, pl.ds(16 * d, 16)]
                pk_v[pl.ds(e, 16)] = _lane_total(ap, mb_v)
                nk_v[pl.ds(e, 16)] = _lane_total(an, nb_v)
            pvec = pk_v[pl.ds(0, 16)]
            nvec = nk_v[pl.ds(0, 16)]
            sp = 1.0 / (1.0 + jnp.exp(-pvec))
            lp = _log_newton(sp + 1e-10)
            sn = 1.0 / (1.0 + jnp.exp(nvec))
            ln_ = _log_newton(sn + 1e-10)
            gidx = c * 128 + g * 16 + lane
            contrib = jnp.where(gidx < _VALID_W, lp + _ALPHA * ln_,
                                jnp.zeros((16,), jnp.float32))
            return gacc - contrib

        return lax.fori_loop(0, 8, group_body, acc)

    acc = lax.fori_loop(0, _NCH, chunk_body,
                        jnp.zeros((16,), jnp.float32))
    acc_v[...] = acc
    pltpu.sync_copy(acc_v, out_hbm.at[wid])


@jax.jit
def _sc_loss(user_embs, item_embs, edge_u, edge_i, pm, nj):
    mesh = plsc.VectorSubcoreMesh(core_axis_name="c", subcore_axis_name="s")
    f = pl.kernel(
        _sc_body,
        out_type=jax.ShapeDtypeStruct((_NW, 16), jnp.float32),
        mesh=mesh,
        scratch_types=[
            pltpu.VMEM((_NCH, _CHUNK), jnp.int32),
            pltpu.VMEM((_NCH, _CHUNK), jnp.int32),
            pltpu.VMEM((_CHUNK,), jnp.int32),
            pltpu.VMEM((_CHUNK,), jnp.int32),
            pltpu.VMEM((_CHUNK, _D), jnp.float32),
            pltpu.VMEM((_CHUNK, _D), jnp.float32),
            pltpu.VMEM((_CHUNK, _D), jnp.float32),
            pltpu.VMEM((16,), jnp.float32),
            pltpu.VMEM((32,), jnp.float32),
            pltpu.VMEM((32,), jnp.float32),
            pltpu.VMEM((32,), jnp.float32),
            pltpu.VMEM((32,), jnp.float32),
            pltpu.SemaphoreType.DMA,
            pltpu.SemaphoreType.DMA,
            pltpu.SemaphoreType.DMA,
            pltpu.SemaphoreType.DMA,
            pltpu.SemaphoreType.DMA,
            pltpu.SemaphoreType.DMA,
            pltpu.SemaphoreType.DMA,
            pltpu.SemaphoreType.DMA,
        ],
    )
    partials = f(user_embs, item_embs, edge_u, edge_i, pm, nj)
    return jnp.sum(partials)


def kernel(user_embs, item_embs, edge_u, edge_i):
    pm_np, nj_np = _sample_constants()
    pm = jnp.asarray(pm_np)
    nj = jnp.asarray(nj_np)
    return _sc_loss(user_embs, item_embs,
                    edge_u.astype(jnp.int32), edge_i.astype(jnp.int32),
                    pm, nj)
